# baseline (speedup 1.0000x reference)
import functools

import jax
import jax.numpy as jnp
from jax import lax
from jax.experimental import pallas as pl
from jax.experimental.pallas import tpu as pltpu
from jax.experimental.pallas import tpu_sc as plsc

B = 16384
NC, NS = 2, 16
NW = NC * NS
BPW = B // NW
CHUNK = 128
NCHUNK = BPW // CHUNK
SPAD = 16

_mesh = plsc.VectorSubcoreMesh(core_axis_name="c", subcore_axis_name="s")


@functools.partial(
    pl.kernel,
    mesh=_mesh,
    compiler_params=pltpu.CompilerParams(use_tc_tiling_on_sc=False),
    out_type=(
        jax.ShapeDtypeStruct((B, 32), jnp.float32),
        jax.ShapeDtypeStruct((B, 32), jnp.float32),
        jax.ShapeDtypeStruct((5, B, SPAD), jnp.float32),
    ),
    scratch_types=(
        [pltpu.VMEM((NCHUNK, CHUNK), jnp.int32) for _ in range(7)]
        + [
            pltpu.VMEM((BPW, 32), jnp.float32),
            pltpu.VMEM((BPW, 32), jnp.float32),
        ]
        + [pltpu.VMEM((BPW, SPAD), jnp.float32) for _ in range(5)]
        + [pltpu.SemaphoreType.DMA]
    ),
)
def _sc_gather(user_table, video_table, t_age, t_gen, t_prov, t_city, t_dev,
               uid, vid, aid, gid, pid, cid, did,
               out_u, out_v, out_s,
               ixu, ixv, ix0, ix1, ix2, ix3, ix4,
               ru, rv, rs0, rs1, rs2, rs3, rs4, sem):
    wid = lax.axis_index("s") * NC + lax.axis_index("c")
    base = wid * BPW
    crow = wid * NCHUNK

    idx_refs = (ixu, ixv, ix0, ix1, ix2, ix3, ix4)
    id_hbm = (uid, vid, aid, gid, pid, cid, did)
    tables = (user_table, video_table, t_age, t_gen, t_prov, t_city, t_dev)
    rows = (ru, rv, rs0, rs1, rs2, rs3, rs4)

    for ix, ids in zip(idx_refs, id_hbm):
        pltpu.sync_copy(ids.at[pl.ds(crow, NCHUNK)], ix)

    copies = []
    for ix, tab, dst in zip(idx_refs, tables, rows):
        for j in range(NCHUNK):
            copies.append(
                pltpu.async_copy(tab.at[ix.at[j]],
                                 dst.at[pl.ds(j * CHUNK, CHUNK)], sem))
    for c in copies:
        c.wait()

    pltpu.sync_copy(ru, out_u.at[pl.ds(base, BPW)])
    pltpu.sync_copy(rv, out_v.at[pl.ds(base, BPW)])
    for t, rs in enumerate((rs0, rs1, rs2, rs3, rs4)):
        pltpu.sync_copy(rs, out_s.at[t, pl.ds(base, BPW)])


def _tc_body(u_ref, v_ref, s_ref, vsc_ref, vact_ref, vdir_ref, sco_ref,
             dur_ref, Wsec_ref, Wact_ref, Wdir_ref, Wsco_ref, Wdur_ref,
             W1u_ref, W1v_ref, W1sec_ref, W1act_ref, W1dir_ref, w1sco_ref,
             w1dur_ref, W1s_ref, bsec_ref, bact_ref, bdir_ref, bsco_ref,
             bdur_ref, b1_ref, W2_ref, b2_ref, W3_ref, b3_ref, out_ref):
    f32 = jnp.float32
    dot = functools.partial(jnp.dot, preferred_element_type=f32)

    h = dot(u_ref[...], W1u_ref[...])
    h += dot(v_ref[...], W1v_ref[...])
    h += dot(vsc_ref[...], dot(Wsec_ref[...], W1sec_ref[...]))
    h += dot(vact_ref[...], dot(Wact_ref[...], W1act_ref[...]))
    h += dot(vdir_ref[...], dot(Wdir_ref[...], W1dir_ref[...]))
    h += dot(sco_ref[...], dot(Wsco_ref[...], w1sco_ref[...]))
    h += dot(dur_ref[...], dot(Wdur_ref[...], w1dur_ref[...]))
    for t in range(5):
        h += dot(s_ref[t], W1s_ref[t])
    bias = b1_ref[...]
    bias += dot(bsec_ref[...], W1sec_ref[...])
    bias += dot(bact_ref[...], W1act_ref[...])
    bias += dot(bdir_ref[...], W1dir_ref[...])
    bias += dot(bsco_ref[...], w1sco_ref[...])
    bias += dot(bdur_ref[...], w1dur_ref[...])
    h = jnp.maximum(h + bias, 0.0)
    h = jnp.maximum(dot(h, W2_ref[...]) + b2_ref[...], 0.0)
    out_ref[...] = dot(h, W3_ref[...]) + b3_ref[...]


def kernel(user_id, video_id, video_second_class, video_actor_list,
           video_director_list, video_score, video_duration, age, gender,
           province, city_level, device_name, user_table, video_table,
           age_table, gender_table, province_table, city_table, device_table,
           W_sec, b_sec, W_act, b_act, W_dir, b_dir, W_score, b_score,
           W_dur, b_dur, W_fc1, b_fc1, W_fc2, b_fc2, W_out, b_out):
    i32 = jnp.int32
    f32 = jnp.float32

    ids2d = [x.astype(i32).reshape(B // CHUNK, CHUNK)
             for x in (user_id, video_id, age, gender, province, city_level,
                       device_name)]
    pad = lambda t: jnp.pad(t, ((0, 0), (0, SPAD - t.shape[1])))
    small_tabs = [pad(t) for t in (age_table, gender_table, province_table,
                                   city_table, device_table)]

    out_u, out_v, out_s = _sc_gather(user_table, video_table, *small_tabs,
                                     *ids2d)

    W1u = W_fc1[0:32]
    W1v = W_fc1[32:64]
    W1sec = W_fc1[64:69]
    W1act = W_fc1[69:74]
    W1dir = W_fc1[74:79]
    w1sco = W_fc1[79:80]
    w1dur = W_fc1[80:81]
    W1s = jnp.stack([jnp.pad(W_fc1[81 + 5 * t:86 + 5 * t],
                             ((0, SPAD - 5), (0, 0))) for t in range(5)])

    blk = 2048
    grid = (B // blk,)
    row_spec = lambda w: pl.BlockSpec((blk, w), lambda i: (i, 0))
    full2 = lambda a: pl.BlockSpec(a.shape, lambda i: (0,) * a.ndim)

    sco = video_score.reshape(B, 1).astype(f32)
    dur = video_duration.reshape(B, 1).astype(f32)
    b_sec2 = b_sec.reshape(1, 5)
    b_act2 = b_act.reshape(1, 5)
    b_dir2 = b_dir.reshape(1, 5)
    b_sco2 = b_score.reshape(1, 1)
    b_dur2 = b_dur.reshape(1, 1)
    b12 = b_fc1.reshape(1, 64)
    b22 = b_fc2.reshape(1, 32)
    b32 = b_out.reshape(1, 10)

    ins = (out_u, out_v, out_s, video_second_class, video_actor_list,
           video_director_list, sco, dur, W_sec, W_act, W_dir, W_score,
           W_dur, W1u, W1v, W1sec, W1act, W1dir, w1sco, w1dur, W1s,
           b_sec2, b_act2, b_dir2, b_sco2, b_dur2, b12, W_fc2, b22,
           W_out, b32)
    in_specs = [
        row_spec(32), row_spec(32),
        pl.BlockSpec((5, blk, SPAD), lambda i: (0, i, 0)),
        row_spec(5), row_spec(5), row_spec(5), row_spec(1), row_spec(1),
    ] + [full2(a) for a in ins[8:]]

    return pl.pallas_call(
        _tc_body,
        grid=grid,
        in_specs=in_specs,
        out_specs=pl.BlockSpec((blk, 10), lambda i: (i, 0)),
        out_shape=jax.ShapeDtypeStruct((B, 10), f32),
    )(*ins)
